# baseline (speedup 1.0000x reference)
import functools

import jax
import jax.numpy as jnp
from jax import lax
from jax.experimental import pallas as pl
from jax.experimental.pallas import tpu as pltpu
from jax.experimental.pallas import tpu_sc as plsc

N = 10000
E = 320000
F = 128
H = 256
D = 128

NC = 2
NS = 16
L = 16
CH = 200
CHS = 400

RPT_A = 632
RPT_LAST = N - (NS - 1) * RPT_A

_SC_MESH = plsc.VectorSubcoreMesh(core_axis_name="c", subcore_axis_name="s")


def _tile_rows(s, fn):
    @pl.when(s < NS - 1)
    def _():
        fn(pl.multiple_of(s * RPT_A, 8), RPT_A)

    @pl.when(s == NS - 1)
    def _():
        fn((NS - 1) * RPT_A, RPT_LAST)


@functools.partial(
    pl.kernel,
    out_type=(jax.ShapeDtypeStruct((N,), jnp.float32),
              jax.ShapeDtypeStruct((N,), jnp.float32)),
    mesh=_SC_MESH,
    scratch_types=[
        pltpu.VMEM((CHS,), jnp.int32),
        pltpu.VMEM((CHS,), jnp.float32),
        pltpu.VMEM_SHARED((N,), jnp.float32),
    ],
)
def _deg_kernel(dst_hbm, zeros_hbm, out0_hbm, out1_hbm, idx_v, ones_v, acc):
    c = lax.axis_index("c")
    s = lax.axis_index("s")
    ept = E // (NC * NS)

    @pl.when(s == 0)
    def _():
        pltpu.sync_copy(zeros_hbm, acc)

    for j in range(CHS // L):
        ones_v[pl.ds(j * L, L)] = jnp.ones((L,), jnp.float32)
    plsc.subcore_barrier()

    base = (c * NS + s) * ept

    def chunk(g, _):
        pltpu.sync_copy(dst_hbm.at[pl.ds(base + g * CHS, CHS)], idx_v)
        pltpu.sync_copy(ones_v, acc.at[idx_v], add=True)
        return ()

    lax.fori_loop(0, ept // CHS, chunk, ())
    plsc.subcore_barrier()

    @pl.when((s == 0) & (c == 0))
    def _():
        pltpu.sync_copy(acc, out0_hbm)

    @pl.when((s == 0) & (c == 1))
    def _():
        pltpu.sync_copy(acc, out1_hbm)


_EPT_COL = E // NS


@functools.partial(
    pl.kernel,
    out_type=jax.ShapeDtypeStruct((N, 256), jnp.float32),
    mesh=_SC_MESH,
    scratch_types=[
        pltpu.VMEM((CH,), jnp.int32),
        pltpu.VMEM((CH,), jnp.int32),
        pltpu.VMEM((CH, 128), jnp.float32),
        pltpu.VMEM_SHARED((N, 128), jnp.float32),
        pltpu.SemaphoreType.DMA,
    ],
)
def _colsplit_scatter(table_hbm, src_hbm, dst_hbm, zeros_hbm, out_hbm,
                      idx_s, idx_d, rows, acc, sem):
    c = lax.axis_index("c")
    s = lax.axis_index("s")

    def zero(off, sz):
        pltpu.sync_copy(zeros_hbm.at[pl.ds(off, sz)], acc.at[pl.ds(off, sz)])

    _tile_rows(s, zero)
    plsc.subcore_barrier()
    base = s * _EPT_COL

    def chunk(g, _):
        off = base + g * CH
        pltpu.sync_copy(src_hbm.at[pl.ds(c * E + off, CH)], idx_s)
        pltpu.sync_copy(dst_hbm.at[pl.ds(off, CH)], idx_d)
        pltpu.async_copy(table_hbm.at[idx_s], rows, sem).wait()
        pltpu.sync_copy(rows, acc.at[idx_d], add=True)
        return ()

    lax.fori_loop(0, _EPT_COL // CH, chunk, ())
    plsc.subcore_barrier()

    def wb(off, sz):
        pltpu.sync_copy(
            acc.at[pl.ds(off, sz)],
            out_hbm.at[pl.ds(off, sz), pl.ds(pl.multiple_of(c * 128, 128), 128)])

    _tile_rows(s, wb)


_EPT_EDGE = E // (NC * NS)


@functools.partial(
    pl.kernel,
    out_type=(jax.ShapeDtypeStruct((N, 128), jnp.float32),
              jax.ShapeDtypeStruct((N, 128), jnp.float32)),
    mesh=_SC_MESH,
    scratch_types=[
        pltpu.VMEM((CH,), jnp.int32),
        pltpu.VMEM((CH,), jnp.int32),
        pltpu.VMEM((CH, 128), jnp.float32),
        pltpu.VMEM_SHARED((N, 128), jnp.float32),
        pltpu.SemaphoreType.DMA,
    ],
)
def _edgesplit_scatter(table_hbm, src_hbm, dst_hbm, zeros_hbm,
                       out0_hbm, out1_hbm, idx_s, idx_d, rows, acc, sem):
    c = lax.axis_index("c")
    s = lax.axis_index("s")

    def zero(off, sz):
        pltpu.sync_copy(zeros_hbm.at[pl.ds(off, sz)], acc.at[pl.ds(off, sz)])

    _tile_rows(s, zero)
    plsc.subcore_barrier()
    base = (c * NS + s) * _EPT_EDGE

    def chunk(g, _):
        off = base + g * CH
        pltpu.sync_copy(src_hbm.at[pl.ds(off, CH)], idx_s)
        pltpu.sync_copy(dst_hbm.at[pl.ds(off, CH)], idx_d)
        pltpu.async_copy(table_hbm.at[idx_s], rows, sem).wait()
        pltpu.sync_copy(rows, acc.at[idx_d], add=True)
        return ()

    lax.fori_loop(0, _EPT_EDGE // CH, chunk, ())
    plsc.subcore_barrier()

    def wb0(off, sz):
        pltpu.sync_copy(acc.at[pl.ds(off, sz)], out0_hbm.at[pl.ds(off, sz)])

    def wb1(off, sz):
        pltpu.sync_copy(acc.at[pl.ds(off, sz)], out1_hbm.at[pl.ds(off, sz)])

    @pl.when(c == 0)
    def _():
        _tile_rows(s, wb0)

    @pl.when(c == 1)
    def _():
        _tile_rows(s, wb1)


@functools.partial(
    pl.kernel,
    out_type=jax.ShapeDtypeStruct((NC, E, 128), jnp.float32),
    mesh=_SC_MESH,
    scratch_types=[
        pltpu.VMEM((CH,), jnp.int32),
        pltpu.VMEM((CH,), jnp.int32),
        pltpu.VMEM((CH, 128), jnp.float32),
        pltpu.VMEM((CH, 128), jnp.float32),
        pltpu.SemaphoreType.DMA,
        pltpu.SemaphoreType.DMA,
    ],
)
def _gat_pass1(xl2_hbm, xr2_hbm, src_hbm, dst_hbm, g_hbm,
               idx_s, idx_d, rows, rows2, sem, sem2):
    c = lax.axis_index("c")
    s = lax.axis_index("s")
    base = s * _EPT_COL

    def chunk(g, _):
        off = base + g * CH
        pltpu.sync_copy(src_hbm.at[pl.ds(c * E + off, CH)], idx_s)
        pltpu.sync_copy(dst_hbm.at[pl.ds(c * E + off, CH)], idx_d)
        cp1 = pltpu.async_copy(xl2_hbm.at[idx_s], rows, sem)
        cp2 = pltpu.async_copy(xr2_hbm.at[idx_d], rows2, sem2)
        cp1.wait()
        cp2.wait()

        def vadd(e, _):
            for k in range(128 // L):
                sl = pl.ds(k * L, L)
                rows[e, sl] = rows[e, sl] + rows2[e, sl]
            return ()

        lax.fori_loop(0, CH, vadd, ())
        pltpu.sync_copy(rows, g_hbm.at[c, pl.ds(off, CH)])
        return ()

    lax.fori_loop(0, _EPT_COL // CH, chunk, ())


@functools.partial(
    pl.kernel,
    out_type=(jax.ShapeDtypeStruct((N,), jnp.float32),
              jax.ShapeDtypeStruct((N,), jnp.float32)),
    mesh=_SC_MESH,
    scratch_types=[
        pltpu.VMEM((CH,), jnp.int32),
        pltpu.VMEM((CH,), jnp.float32),
        pltpu.VMEM_SHARED((N,), jnp.float32),
    ],
)
def _val_segsum(vals_hbm, dst_hbm, zeros_hbm, out0_hbm, out1_hbm,
                idx_v, val_v, acc):
    c = lax.axis_index("c")
    s = lax.axis_index("s")

    @pl.when(s == 0)
    def _():
        pltpu.sync_copy(zeros_hbm, acc)

    plsc.subcore_barrier()
    base = (c * NS + s) * _EPT_EDGE

    def chunk(g, _):
        off = base + g * CH
        pltpu.sync_copy(dst_hbm.at[pl.ds(off, CH)], idx_v)
        pltpu.sync_copy(vals_hbm.at[pl.ds(off, CH)], val_v)
        pltpu.sync_copy(val_v, acc.at[idx_v], add=True)
        return ()

    lax.fori_loop(0, _EPT_EDGE // CH, chunk, ())
    plsc.subcore_barrier()

    @pl.when((s == 0) & (c == 0))
    def _():
        pltpu.sync_copy(acc, out0_hbm)

    @pl.when((s == 0) & (c == 1))
    def _():
        pltpu.sync_copy(acc, out1_hbm)


@functools.partial(
    pl.kernel,
    out_type=jax.ShapeDtypeStruct((N, 256), jnp.float32),
    mesh=_SC_MESH,
    scratch_types=[
        pltpu.VMEM((CH,), jnp.int32),
        pltpu.VMEM((CH,), jnp.int32),
        pltpu.VMEM((CH * L,), jnp.float32),
        pltpu.VMEM((CH, 128), jnp.float32),
        pltpu.VMEM_SHARED((N, 128), jnp.float32),
        pltpu.SemaphoreType.DMA,
    ],
)
def _colsplit_scatter_w(table_hbm, src_hbm, dst_hbm, w16_hbm, zeros_hbm, out_hbm,
                        idx_s, idx_d, w_v, rows, acc, sem):
    c = lax.axis_index("c")
    s = lax.axis_index("s")

    def zero(off, sz):
        pltpu.sync_copy(zeros_hbm.at[pl.ds(off, sz)], acc.at[pl.ds(off, sz)])

    _tile_rows(s, zero)
    plsc.subcore_barrier()
    base = s * _EPT_COL

    def chunk(g, _):
        off = base + g * CH
        pltpu.sync_copy(src_hbm.at[pl.ds(c * E + off, CH)], idx_s)
        pltpu.sync_copy(dst_hbm.at[pl.ds(off, CH)], idx_d)
        pltpu.sync_copy(w16_hbm.at[pl.ds(off * L, CH * L)], w_v)
        pltpu.async_copy(table_hbm.at[idx_s], rows, sem).wait()

        def scale(e, _):
            wv = w_v[pl.ds(e * L, L)]
            for k in range(128 // L):
                sl = pl.ds(k * L, L)
                rows[e, sl] = rows[e, sl] * wv
            return ()

        lax.fori_loop(0, CH, scale, ())
        pltpu.sync_copy(rows, acc.at[idx_d], add=True)
        return ()

    lax.fori_loop(0, _EPT_COL // CH, chunk, ())
    plsc.subcore_barrier()

    def wb(off, sz):
        pltpu.sync_copy(
            acc.at[pl.ds(off, sz)],
            out_hbm.at[pl.ds(off, sz), pl.ds(pl.multiple_of(c * 128, 128), 128)])

    _tile_rows(s, wb)


def _sc_gcn_scatter(xw_scaled, src2, dst):
    w = xw_scaled.shape[1]
    zeros = jnp.zeros((N, 128), jnp.float32)
    if w == 256:
        table2 = jnp.concatenate([xw_scaled[:, :128], xw_scaled[:, 128:]], axis=0)
        return _colsplit_scatter(table2, src2, dst, zeros)
    assert w == 128
    p0, p1 = _edgesplit_scatter(xw_scaled, src2[:E], dst, zeros)
    return p0 + p1


_BM = 1000
_BE = 2000


def _row_spec(bn, w):
    return pl.BlockSpec((bn, w), lambda i: (i, 0))


def _full_spec(a, b):
    return pl.BlockSpec((a, b), lambda i: (0, 0))


def _k_prep_body(deg0, deg1, x, w, xwp, dis):
    d = lax.rsqrt(deg0[...] + deg1[...] + 1.0)
    xwp[...] = (x[...] @ w[...]) * d
    dis[...] = d


def _tc_prep(deg0, deg1, x, w):
    return pl.pallas_call(
        _k_prep_body,
        grid=(N // _BM,),
        in_specs=[_row_spec(_BM, 1), _row_spec(_BM, 1), _row_spec(_BM, F),
                  _full_spec(F, H)],
        out_specs=[_row_spec(_BM, H), _row_spec(_BM, 1)],
        out_shape=[jax.ShapeDtypeStruct((N, H), jnp.float32),
                   jax.ShapeDtypeStruct((N, 1), jnp.float32)],
    )(deg0, deg1, x, w)


def _k_h_body(scat, xwp, dis, gb, wl, bl, wr, br, wer, att, mea, xl_o, xr_o, ls_o):
    h = jnp.maximum(dis[...] * (scat[...] + xwp[...]) + gb[...], 0.0)
    xl = h @ wl[...] + bl[...]
    xr = h @ wr[...] + br[...]
    m = xl + xr + (mea[0, 0] * (1.0 / E)) * wer[...]
    m = jnp.maximum(m, 0.2 * m)
    ls_o[...] = m @ att[...]
    xl_o[...] = xl
    xr_o[...] = xr


def _tc_h(scat, xwp, dis, gb, wl, bl, wr, br, wer, att, ea_sum):
    return pl.pallas_call(
        _k_h_body,
        grid=(N // _BM,),
        in_specs=[_row_spec(_BM, H), _row_spec(_BM, H), _row_spec(_BM, 1),
                  _full_spec(1, H), _full_spec(H, H), _full_spec(1, H),
                  _full_spec(H, H), _full_spec(1, H), _full_spec(1, H),
                  _full_spec(H, 1), _full_spec(1, 1)],
        out_specs=[_row_spec(_BM, H), _row_spec(_BM, H), _row_spec(_BM, 1)],
        out_shape=[jax.ShapeDtypeStruct((N, H), jnp.float32),
                   jax.ShapeDtypeStruct((N, H), jnp.float32),
                   jax.ShapeDtypeStruct((N, 1), jnp.float32)],
    )(scat, xwp, dis, gb, wl, bl, wr, br, wer, att, ea_sum)


def _k_sum_body(v, o):
    @pl.when(pl.program_id(0) == 0)
    def _():
        o[...] = jnp.zeros_like(o)

    o[...] = o[...] + jnp.sum(v[...]).reshape(1, 1)


def _tc_sum(v):
    n = v.shape[0]
    return pl.pallas_call(
        _k_sum_body,
        grid=(n // _BE,),
        in_specs=[_row_spec(_BE, 1)],
        out_specs=_full_spec(1, 1),
        out_shape=jax.ShapeDtypeStruct((1, 1), jnp.float32),
    )(v)


def _k_logits_body(g0, g1, ea, wer, att, o):
    m = jnp.concatenate([g0[0], g1[0]], axis=-1) + ea[...] * wer[...]
    m = jnp.maximum(m, 0.2 * m)
    o[...] = m @ att[...]


def _tc_logits(G, ea, wer, att):
    return pl.pallas_call(
        _k_logits_body,
        grid=(E // _BE,),
        in_specs=[pl.BlockSpec((1, _BE, 128), lambda i: (0, i, 0)),
                  pl.BlockSpec((1, _BE, 128), lambda i: (1, i, 0)),
                  _row_spec(_BE, 1), _full_spec(1, H), _full_spec(H, 1)],
        out_specs=_row_spec(_BE, 1),
        out_shape=jax.ShapeDtypeStruct((E, 1), jnp.float32),
    )(G, G, ea, wer, att)


def _k_max_body(v, o):
    m = jnp.max(v[...]).reshape(1, 1)

    @pl.when(pl.program_id(0) == 0)
    def _():
        o[...] = m

    o[...] = jnp.maximum(o[...], m)


def _tc_max(v):
    n = v.shape[0]
    return pl.pallas_call(
        _k_max_body,
        grid=(n // _BE,),
        in_specs=[_row_spec(_BE, 1)],
        out_specs=_full_spec(1, 1),
        out_shape=jax.ShapeDtypeStruct((1, 1), jnp.float32),
    )(v)


def _k_exp_body(l, c0, ex_o, ex16_o):
    ex = jnp.exp(l[...] - c0[0, 0])
    ex_o[...] = ex
    ex16_o[...] = jnp.broadcast_to(ex, (ex.shape[0], 16))


def _tc_exp(logit, c0):
    return pl.pallas_call(
        _k_exp_body,
        grid=(E // _BE,),
        in_specs=[_row_spec(_BE, 1), _full_spec(1, 1)],
        out_specs=[_row_spec(_BE, 1), _row_spec(_BE, 16)],
        out_shape=[jax.ShapeDtypeStruct((E, 1), jnp.float32),
                   jax.ShapeDtypeStruct((E, 16), jnp.float32)],
    )(logit, c0)


def _k_gatout_body(num, xl, ls, c0, d0, d1, gatb, dis, w2, o):
    es = jnp.exp(ls[...] - c0[0, 0])
    den = d0[...] + d1[...] + es
    h2 = jnp.maximum((num[...] + es * xl[...]) / den + gatb[...], 0.0)
    o[...] = (h2 @ w2[...]) * dis[...]


def _tc_gatout(num, xl, ls, c0, d0, d1, gatb, dis, w2):
    return pl.pallas_call(
        _k_gatout_body,
        grid=(N // _BM,),
        in_specs=[_row_spec(_BM, H), _row_spec(_BM, H), _row_spec(_BM, 1),
                  _full_spec(1, 1), _row_spec(_BM, 1), _row_spec(_BM, 1),
                  _full_spec(1, H), _row_spec(_BM, 1), _full_spec(H, D)],
        out_specs=_row_spec(_BM, D),
        out_shape=jax.ShapeDtypeStruct((N, D), jnp.float32),
    )(num, xl, ls, c0, d0, d1, gatb, dis, w2)


def _k_final_body(q0, q1, hw, dis, g2b, w, b, o):
    z = dis[...] * (q0[...] + q1[...] + hw[...]) + g2b[...]
    o[...] = jnp.maximum(z @ w[...] + b[...], 0.0)


def _tc_final(q0, q1, hw, dis, g2b, w, b):
    return pl.pallas_call(
        _k_final_body,
        grid=(N // _BM,),
        in_specs=[_row_spec(_BM, D), _row_spec(_BM, D), _row_spec(_BM, D),
                  _row_spec(_BM, 1), _full_spec(1, D), _full_spec(D, H),
                  _full_spec(1, H)],
        out_specs=_row_spec(_BM, H),
        out_shape=jax.ShapeDtypeStruct((N, H), jnp.float32),
    )(q0, q1, hw, dis, g2b, w, b)


def _decoder_mm_body(nx_a, nx_b, o_ref):
    a = nx_a[...]
    b = nx_b[...]
    o_ref[...] = jax.lax.dot_general(
        a, b, (((1,), (1,)), ((), ())), preferred_element_type=jnp.float32)


def _decoder_matmul(node_x):
    TM = 1024
    grid = (pl.cdiv(N, TM), pl.cdiv(N, TM))
    return pl.pallas_call(
        _decoder_mm_body,
        grid=grid,
        in_specs=[
            pl.BlockSpec((TM, H), lambda i, j: (i, 0)),
            pl.BlockSpec((TM, H), lambda i, j: (j, 0)),
        ],
        out_specs=pl.BlockSpec((TM, TM), lambda i, j: (i, j)),
        out_shape=jax.ShapeDtypeStruct((N, N), jnp.float32),
    )(node_x, node_x)


def _halves(a):
    return jnp.concatenate([a[:, :128], a[:, 128:]], axis=0)


def kernel(x, edge_index, edge_attr, batch, gcn_W, gcn_b, gat_Wl, gat_bl,
           gat_Wr, gat_br, gat_We, gat_att, gat_b, gcn2_W, gcn2_b, lin1_W, lin1_b):
    src = edge_index[0]
    dst = edge_index[1]
    src2 = jnp.concatenate([src, src + N])
    dst2 = jnp.concatenate([dst, dst + N])
    zeros1 = jnp.zeros((N,), jnp.float32)
    zeros = jnp.zeros((N, 128), jnp.float32)
    ea = edge_attr

    deg0, deg1 = _deg_kernel(dst, zeros1)
    xwp, dis = _tc_prep(deg0[:, None], deg1[:, None], x, gcn_W)
    scat1 = _colsplit_scatter(_halves(xwp), src2, dst, zeros)

    ea_sum = _tc_sum(ea)
    xl, xr, logit_self = _tc_h(scat1, xwp, dis, gcn_b[None, :], gat_Wl,
                               gat_bl[None, :], gat_Wr, gat_br[None, :],
                               gat_We, gat_att[:, None], ea_sum)
    xl2 = _halves(xl)
    G = _gat_pass1(xl2, _halves(xr), src2, dst2)
    logit = _tc_logits(G, ea, gat_We, gat_att[:, None])
    c0 = _tc_max(jnp.concatenate([logit, logit_self], axis=0))
    expl, expl16 = _tc_exp(logit, c0)
    den0, den1 = _val_segsum(expl.reshape(E), dst, zeros1)
    num = _colsplit_scatter_w(xl2, src2, dst, expl16.reshape(E * 16), zeros)
    hw2p = _tc_gatout(num, xl, logit_self, c0, den0[:, None], den1[:, None],
                      gat_b[None, :], dis, gcn2_W)

    q0, q1 = _edgesplit_scatter(hw2p, src, dst, zeros)
    node_x = _tc_final(q0, q1, hw2p, dis, gcn2_b[None, :], lin1_W,
                       lin1_b[None, :])
    return _decoder_matmul(node_x)


# speedup vs baseline: 1.0859x; 1.0859x over previous
import functools

import jax
import jax.numpy as jnp
from jax import lax
from jax.experimental import pallas as pl
from jax.experimental.pallas import tpu as pltpu
from jax.experimental.pallas import tpu_sc as plsc

N = 10000
E = 320000
F = 128
H = 256
D = 128

NC = 2
NS = 16
L = 16
CH = 200
CHC = 80
CHS = 400

RPT_A = 632
RPT_LAST = N - (NS - 1) * RPT_A

_SC_MESH = plsc.VectorSubcoreMesh(core_axis_name="c", subcore_axis_name="s")


def _db_loop(nch, load_issue, consume):
    load_issue(0, 0)

    def body(gg, _):
        g0 = 2 * gg
        load_issue(g0 + 1, 1)
        consume(g0, 0)

        @pl.when(gg < nch // 2 - 1)
        def _():
            load_issue(g0 + 2, 0)

        consume(g0 + 1, 1)
        return ()

    lax.fori_loop(0, nch // 2, body, ())


def _tile_rows(s, fn):
    @pl.when(s < NS - 1)
    def _():
        fn(pl.multiple_of(s * RPT_A, 8), RPT_A)

    @pl.when(s == NS - 1)
    def _():
        fn((NS - 1) * RPT_A, RPT_LAST)


@functools.partial(
    pl.kernel,
    out_type=(jax.ShapeDtypeStruct((N,), jnp.float32),
              jax.ShapeDtypeStruct((N,), jnp.float32)),
    mesh=_SC_MESH,
    scratch_types=[
        pltpu.VMEM((CHS,), jnp.int32),
        pltpu.VMEM((CHS,), jnp.float32),
        pltpu.VMEM_SHARED((N,), jnp.float32),
    ],
)
def _deg_kernel(dst_hbm, zeros_hbm, out0_hbm, out1_hbm, idx_v, ones_v, acc):
    c = lax.axis_index("c")
    s = lax.axis_index("s")
    ept = E // (NC * NS)

    @pl.when(s == 0)
    def _():
        pltpu.sync_copy(zeros_hbm, acc)

    for j in range(CHS // L):
        ones_v[pl.ds(j * L, L)] = jnp.ones((L,), jnp.float32)
    plsc.subcore_barrier()

    base = (c * NS + s) * ept

    def chunk(g, _):
        pltpu.sync_copy(dst_hbm.at[pl.ds(base + g * CHS, CHS)], idx_v)
        pltpu.sync_copy(ones_v, acc.at[idx_v], add=True)
        return ()

    lax.fori_loop(0, ept // CHS, chunk, ())
    plsc.subcore_barrier()

    @pl.when((s == 0) & (c == 0))
    def _():
        pltpu.sync_copy(acc, out0_hbm)

    @pl.when((s == 0) & (c == 1))
    def _():
        pltpu.sync_copy(acc, out1_hbm)


_EPT_COL = E // NS


@functools.partial(
    pl.kernel,
    out_type=jax.ShapeDtypeStruct((N, 256), jnp.float32),
    mesh=_SC_MESH,
    scratch_types=[
        pltpu.VMEM((CHC,), jnp.int32),
        pltpu.VMEM((CHC,), jnp.int32),
        pltpu.VMEM((CHC,), jnp.int32),
        pltpu.VMEM((CHC,), jnp.int32),
        pltpu.VMEM((CHC, 128), jnp.float32),
        pltpu.VMEM((CHC, 128), jnp.float32),
        pltpu.VMEM_SHARED((N, 128), jnp.float32),
        pltpu.SemaphoreType.DMA,
        pltpu.SemaphoreType.DMA,
    ],
)
def _colsplit_scatter(table_hbm, src_hbm, dst_hbm, zeros_hbm, out_hbm,
                      idx_s0, idx_s1, idx_d0, idx_d1, rows0, rows1,
                      acc, sem0, sem1):
    c = lax.axis_index("c")
    s = lax.axis_index("s")
    idx_s = [idx_s0, idx_s1]
    idx_d = [idx_d0, idx_d1]
    rows = [rows0, rows1]
    sems = [sem0, sem1]

    def zero(off, sz):
        pltpu.sync_copy(zeros_hbm.at[pl.ds(off, sz)], acc.at[pl.ds(off, sz)])

    _tile_rows(s, zero)
    plsc.subcore_barrier()
    base = s * _EPT_COL

    def load_issue(g, b):
        off = base + g * CHC
        pltpu.sync_copy(src_hbm.at[pl.ds(c * E + off, CHC)], idx_s[b])
        pltpu.sync_copy(dst_hbm.at[pl.ds(off, CHC)], idx_d[b])
        pltpu.async_copy(table_hbm.at[idx_s[b]], rows[b], sems[b])

    def consume(g, b):
        pltpu.make_async_copy(table_hbm.at[idx_s[b]], rows[b], sems[b]).wait()
        pltpu.sync_copy(rows[b], acc.at[idx_d[b]], add=True)

    _db_loop(_EPT_COL // CHC, load_issue, consume)
    plsc.subcore_barrier()

    def wb(off, sz):
        pltpu.sync_copy(
            acc.at[pl.ds(off, sz)],
            out_hbm.at[pl.ds(off, sz), pl.ds(pl.multiple_of(c * 128, 128), 128)])

    _tile_rows(s, wb)


_EPT_EDGE = E // (NC * NS)


@functools.partial(
    pl.kernel,
    out_type=(jax.ShapeDtypeStruct((N, 128), jnp.float32),
              jax.ShapeDtypeStruct((N, 128), jnp.float32)),
    mesh=_SC_MESH,
    scratch_types=[
        pltpu.VMEM((CH,), jnp.int32),
        pltpu.VMEM((CH,), jnp.int32),
        pltpu.VMEM((CH, 128), jnp.float32),
        pltpu.VMEM_SHARED((N, 128), jnp.float32),
        pltpu.SemaphoreType.DMA,
    ],
)
def _edgesplit_scatter(table_hbm, src_hbm, dst_hbm, zeros_hbm,
                       out0_hbm, out1_hbm, idx_s, idx_d, rows, acc, sem):
    c = lax.axis_index("c")
    s = lax.axis_index("s")

    def zero(off, sz):
        pltpu.sync_copy(zeros_hbm.at[pl.ds(off, sz)], acc.at[pl.ds(off, sz)])

    _tile_rows(s, zero)
    plsc.subcore_barrier()
    base = (c * NS + s) * _EPT_EDGE

    def chunk(g, _):
        off = base + g * CH
        pltpu.sync_copy(src_hbm.at[pl.ds(off, CH)], idx_s)
        pltpu.sync_copy(dst_hbm.at[pl.ds(off, CH)], idx_d)
        pltpu.async_copy(table_hbm.at[idx_s], rows, sem).wait()
        pltpu.sync_copy(rows, acc.at[idx_d], add=True)
        return ()

    lax.fori_loop(0, _EPT_EDGE // CH, chunk, ())
    plsc.subcore_barrier()

    def wb0(off, sz):
        pltpu.sync_copy(acc.at[pl.ds(off, sz)], out0_hbm.at[pl.ds(off, sz)])

    def wb1(off, sz):
        pltpu.sync_copy(acc.at[pl.ds(off, sz)], out1_hbm.at[pl.ds(off, sz)])

    @pl.when(c == 0)
    def _():
        _tile_rows(s, wb0)

    @pl.when(c == 1)
    def _():
        _tile_rows(s, wb1)


@functools.partial(
    pl.kernel,
    out_type=jax.ShapeDtypeStruct((NC, E, 128), jnp.float32),
    mesh=_SC_MESH,
    scratch_types=[
        pltpu.VMEM((CH,), jnp.int32),
        pltpu.VMEM((CH,), jnp.int32),
        pltpu.VMEM((CH,), jnp.int32),
        pltpu.VMEM((CH,), jnp.int32),
        pltpu.VMEM((CH, 128), jnp.float32),
        pltpu.VMEM((CH, 128), jnp.float32),
        pltpu.VMEM((CH, 128), jnp.float32),
        pltpu.VMEM((CH, 128), jnp.float32),
        pltpu.SemaphoreType.DMA,
        pltpu.SemaphoreType.DMA,
        pltpu.SemaphoreType.DMA,
        pltpu.SemaphoreType.DMA,
    ],
)
def _gat_pass1(xl2_hbm, xr2_hbm, src_hbm, dst_hbm, g_hbm,
               idx_s0, idx_s1, idx_d0, idx_d1, rowsa0, rowsa1,
               rowsb0, rowsb1, semA0, semA1, semB0, semB1):
    c = lax.axis_index("c")
    s = lax.axis_index("s")
    base = s * _EPT_COL
    idx_s = [idx_s0, idx_s1]
    idx_d = [idx_d0, idx_d1]
    rowsa = [rowsa0, rowsa1]
    rowsb = [rowsb0, rowsb1]
    semA = [semA0, semA1]
    semB = [semB0, semB1]

    def load_issue(g, b):
        off = base + g * CH
        pltpu.sync_copy(src_hbm.at[pl.ds(c * E + off, CH)], idx_s[b])
        pltpu.sync_copy(dst_hbm.at[pl.ds(c * E + off, CH)], idx_d[b])
        pltpu.async_copy(xl2_hbm.at[idx_s[b]], rowsa[b], semA[b])
        pltpu.async_copy(xr2_hbm.at[idx_d[b]], rowsb[b], semB[b])

    def consume(g, b):
        off = base + g * CH
        pltpu.make_async_copy(xl2_hbm.at[idx_s[b]], rowsa[b], semA[b]).wait()
        pltpu.make_async_copy(xr2_hbm.at[idx_d[b]], rowsb[b], semB[b]).wait()

        def vadd(e, _):
            for k in range(128 // L):
                sl = pl.ds(k * L, L)
                rowsa[b][e, sl] = rowsa[b][e, sl] + rowsb[b][e, sl]
            return ()

        lax.fori_loop(0, CH, vadd, ())
        pltpu.sync_copy(rowsa[b], g_hbm.at[c, pl.ds(off, CH)])

    _db_loop(_EPT_COL // CH, load_issue, consume)


@functools.partial(
    pl.kernel,
    out_type=(jax.ShapeDtypeStruct((N,), jnp.float32),
              jax.ShapeDtypeStruct((N,), jnp.float32)),
    mesh=_SC_MESH,
    scratch_types=[
        pltpu.VMEM((CH,), jnp.int32),
        pltpu.VMEM((CH,), jnp.float32),
        pltpu.VMEM_SHARED((N,), jnp.float32),
    ],
)
def _val_segsum(vals_hbm, dst_hbm, zeros_hbm, out0_hbm, out1_hbm,
                idx_v, val_v, acc):
    c = lax.axis_index("c")
    s = lax.axis_index("s")

    @pl.when(s == 0)
    def _():
        pltpu.sync_copy(zeros_hbm, acc)

    plsc.subcore_barrier()
    base = (c * NS + s) * _EPT_EDGE

    def chunk(g, _):
        off = base + g * CH
        pltpu.sync_copy(dst_hbm.at[pl.ds(off, CH)], idx_v)
        pltpu.sync_copy(vals_hbm.at[pl.ds(off, CH)], val_v)
        pltpu.sync_copy(val_v, acc.at[idx_v], add=True)
        return ()

    lax.fori_loop(0, _EPT_EDGE // CH, chunk, ())
    plsc.subcore_barrier()

    @pl.when((s == 0) & (c == 0))
    def _():
        pltpu.sync_copy(acc, out0_hbm)

    @pl.when((s == 0) & (c == 1))
    def _():
        pltpu.sync_copy(acc, out1_hbm)


@functools.partial(
    pl.kernel,
    out_type=jax.ShapeDtypeStruct((N, 256), jnp.float32),
    mesh=_SC_MESH,
    scratch_types=[
        pltpu.VMEM((CHC,), jnp.int32),
        pltpu.VMEM((CHC,), jnp.int32),
        pltpu.VMEM((CHC,), jnp.int32),
        pltpu.VMEM((CHC,), jnp.int32),
        pltpu.VMEM((CHC * L,), jnp.float32),
        pltpu.VMEM((CHC * L,), jnp.float32),
        pltpu.VMEM((CHC, 128), jnp.float32),
        pltpu.VMEM((CHC, 128), jnp.float32),
        pltpu.VMEM_SHARED((N, 128), jnp.float32),
        pltpu.SemaphoreType.DMA,
        pltpu.SemaphoreType.DMA,
    ],
)
def _colsplit_scatter_w(table_hbm, src_hbm, dst_hbm, w16_hbm, zeros_hbm, out_hbm,
                        idx_s0, idx_s1, idx_d0, idx_d1, w_v0, w_v1,
                        rows0, rows1, acc, sem0, sem1):
    c = lax.axis_index("c")
    s = lax.axis_index("s")
    idx_s = [idx_s0, idx_s1]
    idx_d = [idx_d0, idx_d1]
    w_v = [w_v0, w_v1]
    rows = [rows0, rows1]
    sems = [sem0, sem1]

    def zero(off, sz):
        pltpu.sync_copy(zeros_hbm.at[pl.ds(off, sz)], acc.at[pl.ds(off, sz)])

    _tile_rows(s, zero)
    plsc.subcore_barrier()
    base = s * _EPT_COL

    def load_issue(g, b):
        off = base + g * CHC
        pltpu.sync_copy(src_hbm.at[pl.ds(c * E + off, CHC)], idx_s[b])
        pltpu.sync_copy(dst_hbm.at[pl.ds(off, CHC)], idx_d[b])
        pltpu.sync_copy(w16_hbm.at[pl.ds(off * L, CHC * L)], w_v[b])
        pltpu.async_copy(table_hbm.at[idx_s[b]], rows[b], sems[b])

    def consume(g, b):
        pltpu.make_async_copy(table_hbm.at[idx_s[b]], rows[b], sems[b]).wait()

        def scale(e, _):
            wv = w_v[b][pl.ds(e * L, L)]
            for k in range(128 // L):
                sl = pl.ds(k * L, L)
                rows[b][e, sl] = rows[b][e, sl] * wv
            return ()

        lax.fori_loop(0, CHC, scale, ())
        pltpu.sync_copy(rows[b], acc.at[idx_d[b]], add=True)

    _db_loop(_EPT_COL // CHC, load_issue, consume)
    plsc.subcore_barrier()

    def wb(off, sz):
        pltpu.sync_copy(
            acc.at[pl.ds(off, sz)],
            out_hbm.at[pl.ds(off, sz), pl.ds(pl.multiple_of(c * 128, 128), 128)])

    _tile_rows(s, wb)


def _sc_gcn_scatter(xw_scaled, src2, dst):
    w = xw_scaled.shape[1]
    zeros = jnp.zeros((N, 128), jnp.float32)
    if w == 256:
        table2 = jnp.concatenate([xw_scaled[:, :128], xw_scaled[:, 128:]], axis=0)
        return _colsplit_scatter(table2, src2, dst, zeros)
    assert w == 128
    p0, p1 = _edgesplit_scatter(xw_scaled, src2[:E], dst, zeros)
    return p0 + p1


_BM = 1000
_BE = 2000


def _row_spec(bn, w):
    return pl.BlockSpec((bn, w), lambda i: (i, 0))


def _full_spec(a, b):
    return pl.BlockSpec((a, b), lambda i: (0, 0))


def _k_prep_body(deg0, deg1, x, w, xwp, dis):
    d = lax.rsqrt(deg0[...] + deg1[...] + 1.0)
    xwp[...] = (x[...] @ w[...]) * d
    dis[...] = d


def _tc_prep(deg0, deg1, x, w):
    return pl.pallas_call(
        _k_prep_body,
        grid=(N // _BM,),
        in_specs=[_row_spec(_BM, 1), _row_spec(_BM, 1), _row_spec(_BM, F),
                  _full_spec(F, H)],
        out_specs=[_row_spec(_BM, H), _row_spec(_BM, 1)],
        out_shape=[jax.ShapeDtypeStruct((N, H), jnp.float32),
                   jax.ShapeDtypeStruct((N, 1), jnp.float32)],
    )(deg0, deg1, x, w)


def _k_h_body(scat, xwp, dis, gb, wl, bl, wr, br, wer, att, mea, xl_o, xr_o, ls_o):
    h = jnp.maximum(dis[...] * (scat[...] + xwp[...]) + gb[...], 0.0)
    xl = h @ wl[...] + bl[...]
    xr = h @ wr[...] + br[...]
    m = xl + xr + (mea[0, 0] * (1.0 / E)) * wer[...]
    m = jnp.maximum(m, 0.2 * m)
    ls_o[...] = m @ att[...]
    xl_o[...] = xl
    xr_o[...] = xr


def _tc_h(scat, xwp, dis, gb, wl, bl, wr, br, wer, att, ea_sum):
    return pl.pallas_call(
        _k_h_body,
        grid=(N // _BM,),
        in_specs=[_row_spec(_BM, H), _row_spec(_BM, H), _row_spec(_BM, 1),
                  _full_spec(1, H), _full_spec(H, H), _full_spec(1, H),
                  _full_spec(H, H), _full_spec(1, H), _full_spec(1, H),
                  _full_spec(H, 1), _full_spec(1, 1)],
        out_specs=[_row_spec(_BM, H), _row_spec(_BM, H), _row_spec(_BM, 1)],
        out_shape=[jax.ShapeDtypeStruct((N, H), jnp.float32),
                   jax.ShapeDtypeStruct((N, H), jnp.float32),
                   jax.ShapeDtypeStruct((N, 1), jnp.float32)],
    )(scat, xwp, dis, gb, wl, bl, wr, br, wer, att, ea_sum)


def _k_sum_body(v, o):
    @pl.when(pl.program_id(0) == 0)
    def _():
        o[...] = jnp.zeros_like(o)

    o[...] = o[...] + jnp.sum(v[...]).reshape(1, 1)


def _tc_sum(v):
    n = v.shape[0]
    return pl.pallas_call(
        _k_sum_body,
        grid=(n // _BE,),
        in_specs=[_row_spec(_BE, 1)],
        out_specs=_full_spec(1, 1),
        out_shape=jax.ShapeDtypeStruct((1, 1), jnp.float32),
    )(v)


def _k_logits_body(g0, g1, ea, wer, att, o):
    m = jnp.concatenate([g0[0], g1[0]], axis=-1) + ea[...] * wer[...]
    m = jnp.maximum(m, 0.2 * m)
    o[...] = m @ att[...]


def _tc_logits(G, ea, wer, att):
    return pl.pallas_call(
        _k_logits_body,
        grid=(E // _BE,),
        in_specs=[pl.BlockSpec((1, _BE, 128), lambda i: (0, i, 0)),
                  pl.BlockSpec((1, _BE, 128), lambda i: (1, i, 0)),
                  _row_spec(_BE, 1), _full_spec(1, H), _full_spec(H, 1)],
        out_specs=_row_spec(_BE, 1),
        out_shape=jax.ShapeDtypeStruct((E, 1), jnp.float32),
    )(G, G, ea, wer, att)


def _k_max_body(v, o):
    m = jnp.max(v[...]).reshape(1, 1)

    @pl.when(pl.program_id(0) == 0)
    def _():
        o[...] = m

    o[...] = jnp.maximum(o[...], m)


def _tc_max(v):
    n = v.shape[0]
    return pl.pallas_call(
        _k_max_body,
        grid=(n // _BE,),
        in_specs=[_row_spec(_BE, 1)],
        out_specs=_full_spec(1, 1),
        out_shape=jax.ShapeDtypeStruct((1, 1), jnp.float32),
    )(v)


def _k_exp_body(l, c0, ex_o, ex16_o):
    ex = jnp.exp(l[...] - c0[0, 0])
    ex_o[...] = ex
    ex16_o[...] = jnp.broadcast_to(ex, (ex.shape[0], 16))


def _tc_exp(logit, c0):
    return pl.pallas_call(
        _k_exp_body,
        grid=(E // _BE,),
        in_specs=[_row_spec(_BE, 1), _full_spec(1, 1)],
        out_specs=[_row_spec(_BE, 1), _row_spec(_BE, 16)],
        out_shape=[jax.ShapeDtypeStruct((E, 1), jnp.float32),
                   jax.ShapeDtypeStruct((E, 16), jnp.float32)],
    )(logit, c0)


def _k_gatout_body(num, xl, ls, c0, d0, d1, gatb, dis, w2, o):
    es = jnp.exp(ls[...] - c0[0, 0])
    den = d0[...] + d1[...] + es
    h2 = jnp.maximum((num[...] + es * xl[...]) / den + gatb[...], 0.0)
    o[...] = (h2 @ w2[...]) * dis[...]


def _tc_gatout(num, xl, ls, c0, d0, d1, gatb, dis, w2):
    return pl.pallas_call(
        _k_gatout_body,
        grid=(N // _BM,),
        in_specs=[_row_spec(_BM, H), _row_spec(_BM, H), _row_spec(_BM, 1),
                  _full_spec(1, 1), _row_spec(_BM, 1), _row_spec(_BM, 1),
                  _full_spec(1, H), _row_spec(_BM, 1), _full_spec(H, D)],
        out_specs=_row_spec(_BM, D),
        out_shape=jax.ShapeDtypeStruct((N, D), jnp.float32),
    )(num, xl, ls, c0, d0, d1, gatb, dis, w2)


def _k_final_body(q0, q1, hw, dis, g2b, w, b, o):
    z = dis[...] * (q0[...] + q1[...] + hw[...]) + g2b[...]
    o[...] = jnp.maximum(z @ w[...] + b[...], 0.0)


def _tc_final(q0, q1, hw, dis, g2b, w, b):
    return pl.pallas_call(
        _k_final_body,
        grid=(N // _BM,),
        in_specs=[_row_spec(_BM, D), _row_spec(_BM, D), _row_spec(_BM, D),
                  _row_spec(_BM, 1), _full_spec(1, D), _full_spec(D, H),
                  _full_spec(1, H)],
        out_specs=_row_spec(_BM, H),
        out_shape=jax.ShapeDtypeStruct((N, H), jnp.float32),
    )(q0, q1, hw, dis, g2b, w, b)


def _decoder_mm_body(nx_a, nx_b, o_ref):
    a = nx_a[...]
    b = nx_b[...]
    o_ref[...] = jax.lax.dot_general(
        a, b, (((1,), (1,)), ((), ())), preferred_element_type=jnp.float32)


def _decoder_matmul(node_x):
    TM = 1024
    grid = (pl.cdiv(N, TM), pl.cdiv(N, TM))
    return pl.pallas_call(
        _decoder_mm_body,
        grid=grid,
        in_specs=[
            pl.BlockSpec((TM, H), lambda i, j: (i, 0)),
            pl.BlockSpec((TM, H), lambda i, j: (j, 0)),
        ],
        out_specs=pl.BlockSpec((TM, TM), lambda i, j: (i, j)),
        out_shape=jax.ShapeDtypeStruct((N, N), jnp.float32),
    )(node_x, node_x)


def _halves(a):
    return jnp.concatenate([a[:, :128], a[:, 128:]], axis=0)


def kernel(x, edge_index, edge_attr, batch, gcn_W, gcn_b, gat_Wl, gat_bl,
           gat_Wr, gat_br, gat_We, gat_att, gat_b, gcn2_W, gcn2_b, lin1_W, lin1_b):
    src = edge_index[0]
    dst = edge_index[1]
    src2 = jnp.concatenate([src, src + N])
    dst2 = jnp.concatenate([dst, dst + N])
    zeros1 = jnp.zeros((N,), jnp.float32)
    zeros = jnp.zeros((N, 128), jnp.float32)
    ea = edge_attr

    deg0, deg1 = _deg_kernel(dst, zeros1)
    xwp, dis = _tc_prep(deg0[:, None], deg1[:, None], x, gcn_W)
    scat1 = _colsplit_scatter(_halves(xwp), src2, dst, zeros)

    ea_sum = _tc_sum(ea)
    xl, xr, logit_self = _tc_h(scat1, xwp, dis, gcn_b[None, :], gat_Wl,
                               gat_bl[None, :], gat_Wr, gat_br[None, :],
                               gat_We, gat_att[:, None], ea_sum)
    xl2 = _halves(xl)
    G = _gat_pass1(xl2, _halves(xr), src2, dst2)
    logit = _tc_logits(G, ea, gat_We, gat_att[:, None])
    c0 = _tc_max(jnp.concatenate([logit, logit_self], axis=0))
    expl, expl16 = _tc_exp(logit, c0)
    den0, den1 = _val_segsum(expl.reshape(E), dst, zeros1)
    num = _colsplit_scatter_w(xl2, src2, dst, expl16.reshape(E * 16), zeros)
    hw2p = _tc_gatout(num, xl, logit_self, c0, den0[:, None], den1[:, None],
                      gat_b[None, :], dis, gcn2_W)

    q0, q1 = _edgesplit_scatter(hw2p, src, dst, zeros)
    node_x = _tc_final(q0, q1, hw2p, dis, gcn2_b[None, :], lin1_W,
                       lin1_b[None, :])
    return _decoder_matmul(node_x)
